# baseline (speedup 1.0000x reference)
import functools

import jax
import jax.numpy as jnp
from jax import lax
from jax.experimental import pallas as pl
from jax.experimental.pallas import tpu as pltpu
from jax.experimental.pallas import tpu_sc as plsc

N_ROWS = 16384
N_COLS = 26
D = 64
B = N_ROWS * N_COLS

NC = 2
NS = 16
NW = NC * NS

B_PER_W = B // NW
CH = 128
N_CHUNKS = B_PER_W // CH
assert N_CHUNKS * CH == B_PER_W


def _body(table_hbm, idx_hbm, out_hbm, idx_v, rows, gsem0, gsem1, ssem0, ssem1):
    c = lax.axis_index("c")
    s = lax.axis_index("s")
    wid = s * NC + c
    base = wid * B_PER_W

    pltpu.sync_copy(idx_hbm.at[wid], idx_v)

    @pl.loop(0, N_CHUNKS, step=2)
    def _(j):
        g0 = pltpu.async_copy(table_hbm.at[idx_v.at[j]], rows.at[0], gsem0)
        g1 = pltpu.async_copy(table_hbm.at[idx_v.at[j + 1]], rows.at[1], gsem1)
        g0.wait()
        s0 = pltpu.async_copy(rows.at[0], out_hbm.at[pl.ds(base + j * CH, CH)], ssem0)
        g1.wait()
        s1 = pltpu.async_copy(rows.at[1], out_hbm.at[pl.ds(base + (j + 1) * CH, CH)], ssem1)
        s0.wait()
        s1.wait()


@jax.jit
def _gather(x_grouped, table):
    mesh = plsc.VectorSubcoreMesh(
        core_axis_name="c", subcore_axis_name="s", num_cores=NC, num_subcores=NS
    )
    run = pl.kernel(
        _body,
        out_type=jax.ShapeDtypeStruct((B, D), jnp.float32),
        mesh=mesh,
        scratch_types=[
            pltpu.VMEM((N_CHUNKS, CH), jnp.int32),
            pltpu.VMEM((2, CH, D), jnp.float32),
            pltpu.SemaphoreType.DMA,
            pltpu.SemaphoreType.DMA,
            pltpu.SemaphoreType.DMA,
            pltpu.SemaphoreType.DMA,
        ],
        compiler_params=pltpu.CompilerParams(use_tc_tiling_on_sc=False),
    )
    return run(table, x_grouped)


def kernel(x, table):
    x_grouped = jnp.reshape(x.astype(jnp.int32), (NW, N_CHUNKS, CH))
    out = _gather(x_grouped, table)
    return jnp.reshape(out, (N_ROWS, N_COLS, D))


# speedup vs baseline: 1.0050x; 1.0050x over previous
import jax
import jax.numpy as jnp
from jax import lax
from jax.experimental import pallas as pl
from jax.experimental.pallas import tpu as pltpu
from jax.experimental.pallas import tpu_sc as plsc

N_ROWS = 16384
N_COLS = 26
D = 64
B = N_ROWS * N_COLS

NC = 2
NS = 16
NW = NC * NS

B_PER_W = B // NW
CH = 128
N_CHUNKS = B_PER_W // CH
assert N_CHUNKS * CH == B_PER_W
assert N_ROWS % CH == 0


def _body(table_hbm, idx_hbm, out_hbm, idx_v, rows, gsem0, gsem1, ssem0, ssem1):
    c = lax.axis_index("c")
    s = lax.axis_index("s")
    wid = s * NC + c
    base = wid * B_PER_W

    pltpu.sync_copy(idx_hbm.at[wid], idx_v)

    @pl.loop(0, N_CHUNKS, step=2)
    def _(k):
        g0 = pltpu.async_copy(table_hbm.at[idx_v.at[k]], rows.at[0], gsem0)
        g1 = pltpu.async_copy(table_hbm.at[idx_v.at[k + 1]], rows.at[1], gsem1)
        p0 = base + k * CH
        j0 = p0 // N_ROWS
        r0 = p0 % N_ROWS
        p1 = p0 + CH
        j1 = p1 // N_ROWS
        r1 = p1 % N_ROWS
        g0.wait()
        s0 = pltpu.async_copy(rows.at[0], out_hbm.at[pl.ds(r0, CH), j0], ssem0)
        g1.wait()
        s1 = pltpu.async_copy(rows.at[1], out_hbm.at[pl.ds(r1, CH), j1], ssem1)
        s0.wait()
        s1.wait()


@jax.jit
def _gather(xt_grouped, table):
    mesh = plsc.VectorSubcoreMesh(
        core_axis_name="c", subcore_axis_name="s", num_cores=NC, num_subcores=NS
    )
    run = pl.kernel(
        _body,
        out_type=jax.ShapeDtypeStruct((N_ROWS, N_COLS, D), jnp.float32),
        mesh=mesh,
        scratch_types=[
            pltpu.VMEM((N_CHUNKS, CH), jnp.int32),
            pltpu.VMEM((2, CH, D), jnp.float32),
            pltpu.SemaphoreType.DMA,
            pltpu.SemaphoreType.DMA,
            pltpu.SemaphoreType.DMA,
            pltpu.SemaphoreType.DMA,
        ],
        compiler_params=pltpu.CompilerParams(use_tc_tiling_on_sc=False),
    )
    return run(table, xt_grouped)


def kernel(x, table):
    xt_grouped = jnp.reshape(jnp.transpose(x).astype(jnp.int32), (NW, N_CHUNKS, CH))
    return _gather(xt_grouped, table)
